# baseline (speedup 1.0000x reference)
import functools

import jax
import jax.numpy as jnp
import numpy as np
from jax import lax
from jax.experimental import pallas as pl
from jax.experimental.pallas import tpu as pltpu
from jax.experimental.pallas import tpu_sc as plsc

N = 10000
E = 320000
D = 128
NC = 2
NS = 16
NW = NC * NS
EPW = E // NW
C = 80
NCHUNK = EPW // C
L = 16
RPT = N // NS
DW = D + L

_mesh = plsc.VectorSubcoreMesh(core_axis_name="c", subcore_axis_name="s")

_GDN = lax.GatherDimensionNumbers(
    offset_dims=(), collapsed_slice_dims=(0,), start_index_map=(0,))


def _perms():
    lane = lax.iota(jnp.int32, L)
    return [jnp.reshape((lane + sh) % L, (L, 1)) for sh in (1, 2, 4, 8)]


def _shuf(x, p):
    return lax.gather(x, p, _GDN, (1,),
                      mode=lax.GatherScatterMode.PROMISE_IN_BOUNDS)


def _lane_sum(x, perms):
    for p in perms:
        x = x + _shuf(x, p)
    return x


def _lane_max(x, perms):
    for p in perms:
        x = jnp.maximum(x, _shuf(x, p))
    return x


def _proj_body(x_ref, wq, bq, wk, bk, wv, bv, ws, bs, q_ref, k_ref, v_ref, s_ref):
    xb = x_ref[...]
    scale = 1.0 / jnp.sqrt(jnp.float32(D))
    q_ref[...] = ((jnp.dot(xb, wq[...], preferred_element_type=jnp.float32)
                   + bq[...]) * scale).astype(jnp.bfloat16)
    k_ref[...] = (jnp.dot(xb, wk[...], preferred_element_type=jnp.float32)
                  + bk[...]).astype(jnp.bfloat16)
    v_ref[...] = (jnp.dot(xb, wv[...], preferred_element_type=jnp.float32)
                  + bv[...]).astype(jnp.bfloat16)
    s_ref[...] = jnp.dot(xb, ws[...], preferred_element_type=jnp.float32) + bs[...]


_BR = 1000


def _proj(x, Wq, bq, Wk, bk, Wv, bv, Ws, bs):
    wspec = pl.BlockSpec((D, D), lambda i: (0, 0))
    bspec = pl.BlockSpec((1, D), lambda i: (0, 0))
    rspec = pl.BlockSpec((_BR, D), lambda i: (i, 0))
    return pl.pallas_call(
        _proj_body,
        grid=(N // _BR,),
        in_specs=[rspec, wspec, bspec, wspec, bspec, wspec, bspec, wspec, bspec],
        out_specs=[rspec, rspec, rspec, rspec],
        out_shape=[jax.ShapeDtypeStruct((N, D), jnp.bfloat16),
                   jax.ShapeDtypeStruct((N, D), jnp.bfloat16),
                   jax.ShapeDtypeStruct((N, D), jnp.bfloat16),
                   jax.ShapeDtypeStruct((N, D), jnp.float32)],
    )(x, Wq, bq.reshape(1, D), Wk, bk.reshape(1, D),
      Wv, bv.reshape(1, D), Ws, bs.reshape(1, D))


@functools.partial(
    pl.kernel,
    out_type=[jax.ShapeDtypeStruct((NW, EPW), jnp.float32),
              jax.ShapeDtypeStruct((NW, L), jnp.float32)],
    mesh=_mesh,
    scratch_types=[
        pltpu.VMEM((NCHUNK, C), jnp.int32),
        pltpu.VMEM((NCHUNK, C), jnp.int32),
        pltpu.VMEM((EPW,), jnp.float32),
        pltpu.VMEM((C, D), jnp.bfloat16),
        pltpu.VMEM((C, D), jnp.bfloat16),
        pltpu.VMEM((C, D), jnp.bfloat16),
        pltpu.VMEM((C, D), jnp.bfloat16),
        pltpu.VMEM((L,), jnp.float32),
        pltpu.SemaphoreType.DMA,
        pltpu.SemaphoreType.DMA,
        pltpu.SemaphoreType.DMA,
        pltpu.SemaphoreType.DMA,
    ],
    compiler_params=pltpu.CompilerParams(use_tc_tiling_on_sc=False,
                                         needs_layout_passes=False),
)
def _edge_logits(qh, kh, src3, dst3, alpha_h, gmax_h,
                 dst2, src2, a_all, q0, k0, q1, k1, m_v,
                 sq0, sk0, sq1, sk1):
    c = lax.axis_index("c")
    s = lax.axis_index("s")
    wid = s * NC + c

    lane = lax.iota(jnp.int32, L)
    perms = _perms()
    qb, kb = (q0, q1), (k0, k1)
    sq, sk = (sq0, sq1), (sk0, sk1)

    pltpu.sync_copy(dst3.at[wid], dst2)
    pltpu.sync_copy(src3.at[wid], src2)

    def issue(g, b):
        pltpu.async_copy(qh.at[dst2.at[g]], qb[b], sq[b])
        pltpu.async_copy(kh.at[src2.at[g]], kb[b], sk[b])

    def wait(g, b):
        pltpu.make_async_copy(qh.at[dst2.at[g]], qb[b], sq[b]).wait()
        pltpu.make_async_copy(kh.at[src2.at[g]], kb[b], sk[b]).wait()

    def compute(g, b, m16):
        q_v, k_v = qb[b], kb[b]

        def group_body(t, m16):
            av = jnp.full((L,), -3e38, jnp.float32)
            for jj in range(L):
                j = t * L + jj
                acc = None
                for r in range(D // (2 * L)):
                    qv = q_v[j, pl.ds(r * 2 * L, 2 * L)]
                    kv = k_v[j, pl.ds(r * 2 * L, 2 * L)]
                    qa, qc = plsc.unpack(qv, format=plsc.PackFormat.INTERLEAVED,
                                         preferred_element_type=jnp.float32)
                    ka, kc = plsc.unpack(kv, format=plsc.PackFormat.INTERLEAVED,
                                         preferred_element_type=jnp.float32)
                    term = qa * ka + qc * kc
                    acc = term if acc is None else acc + term
                av = jnp.where(lane == jj, _lane_sum(acc, perms), av)
            a_all[pl.ds(g * C + t * L, L)] = av
            return jnp.maximum(m16, av)

        return lax.fori_loop(0, C // L, group_body, m16)

    issue(0, 0)

    def dbl_body(i, m16):
        g = 2 * i
        wait(g, 0)
        issue(g + 1, 1)
        m16 = compute(g, 0, m16)
        wait(g + 1, 1)
        issue(g + 2, 0)
        m16 = compute(g + 1, 1, m16)
        return m16

    m16 = lax.fori_loop(0, (NCHUNK - 1) // 2, dbl_body,
                        jnp.full((L,), -3e38, jnp.float32))
    wait(NCHUNK - 1, 0)
    m16 = compute(NCHUNK - 1, 0, m16)

    pltpu.sync_copy(a_all, alpha_h.at[wid])
    m_v[...] = _lane_max(m16, perms)
    pltpu.sync_copy(m_v, gmax_h.at[wid])


@functools.partial(
    pl.kernel,
    out_type=[jax.ShapeDtypeStruct((NC, N, D), jnp.float32),
              jax.ShapeDtypeStruct((NC, N, L), jnp.float32)],
    mesh=_mesh,
    scratch_types=[
        pltpu.VMEM((C,), jnp.int32),
        pltpu.VMEM((C,), jnp.int32),
        pltpu.VMEM((C,), jnp.int32),
        pltpu.VMEM((C,), jnp.int32),
        pltpu.VMEM((C,), jnp.int32),
        pltpu.VMEM((C,), jnp.int32),
        pltpu.VMEM((C,), jnp.float32),
        pltpu.VMEM((C,), jnp.float32),
        pltpu.VMEM((C, D), jnp.bfloat16),
        pltpu.VMEM((C, D), jnp.bfloat16),
        pltpu.VMEM((C, D), jnp.float32),
        pltpu.VMEM((C, D), jnp.float32),
        pltpu.VMEM((C, L), jnp.float32),
        pltpu.VMEM((C, L), jnp.float32),
        pltpu.VMEM((NW, L), jnp.float32),
        pltpu.VMEM_SHARED((N, D), jnp.float32),
        pltpu.VMEM_SHARED((N, L), jnp.float32),
        pltpu.SemaphoreType.DMA,
        pltpu.SemaphoreType.DMA,
        pltpu.SemaphoreType.DMA,
        pltpu.SemaphoreType.DMA,
        pltpu.SemaphoreType.DMA,
        pltpu.SemaphoreType.DMA,
        pltpu.SemaphoreType.DMA,
        pltpu.SemaphoreType.DMA,
        pltpu.SemaphoreType.DMA,
        pltpu.SemaphoreType.DMA,
    ],
    compiler_params=pltpu.CompilerParams(use_tc_tiling_on_sc=False,
                                         needs_layout_passes=False),
)
def _edge_scatter(vh, srch, dsth, alphah, gmaxh, zmsgh, zdenh,
                  msg_out, den_out,
                  dst0, dst1, srcb0, srcb1, dsc0, dsc1, w0, w1, vr0, vr1,
                  v0, v1, d0, d1, gbuf, msg_sh, den_sh,
                  si0, si1, sw0, sw1, gv0, gv1, sm0, sm1, sd0, sd1):
    c = lax.axis_index("c")
    s = lax.axis_index("s")
    wid = s * NC + c
    ebase = wid * EPW
    rbase = s * RPT

    dstb, srcb, dscb = (dst0, dst1), (srcb0, srcb1), (dsc0, dsc1)
    wb, vrb, vb, db = (w0, w1), (vr0, vr1), (v0, v1), (d0, d1)
    si, sw, gv = (si0, si1), (sw0, sw1), (gv0, gv1)
    sm, sd = (sm0, sm1), (sd0, sd1)

    pltpu.sync_copy(zmsgh.at[pl.ds(rbase, RPT)], msg_sh.at[pl.ds(rbase, RPT)])
    pltpu.sync_copy(zdenh.at[pl.ds(rbase, RPT)], den_sh.at[pl.ds(rbase, RPT)])

    pltpu.sync_copy(gmaxh, gbuf)
    m16 = gbuf[0, pl.ds(0, L)]
    for r in range(1, NW):
        m16 = jnp.maximum(m16, gbuf[r, pl.ds(0, L)])
    gmax = _lane_max(m16, _perms())

    plsc.subcore_barrier()

    def issue_idx(g, b):
        pltpu.async_copy(dsth.at[pl.ds(ebase + g * C, C)], dstb[b], si[b])
        pltpu.async_copy(srch.at[pl.ds(ebase + g * C, C)], srcb[b], si[b])

    def wait_idx(g, b):
        pltpu.make_async_copy(dsth.at[pl.ds(ebase + g * C, C)],
                              dstb[b], si[b]).wait()
        pltpu.make_async_copy(srch.at[pl.ds(ebase + g * C, C)],
                              srcb[b], si[b]).wait()

    def issue_w(g, b):
        pltpu.async_copy(alphah.at[wid].at[pl.ds(g * C, C)], wb[b], sw[b])

    def wait_w(g, b):
        pltpu.make_async_copy(alphah.at[wid].at[pl.ds(g * C, C)],
                              wb[b], sw[b]).wait()

    def issue_gather(g, b):
        pltpu.async_copy(vh.at[srcb[b]], vrb[b], gv[b])

    def wait_gather(g, b):
        pltpu.make_async_copy(vh.at[srcb[b]], vrb[b], gv[b]).wait()

    def issue_scatter(g, b):
        pltpu.async_copy(vb[b], msg_sh.at[dscb[b]], sm[b], add=True)
        pltpu.async_copy(db[b], den_sh.at[dscb[b]], sd[b], add=True)

    def wait_scatter(g, b):
        pltpu.make_async_copy(vb[b], msg_sh.at[dscb[b]], sm[b]).wait()
        pltpu.make_async_copy(db[b], den_sh.at[dscb[b]], sd[b]).wait()

    def copy_dsc(b):
        for t in range(C // L):
            dscb[b][pl.ds(t * L, L)] = dstb[b][pl.ds(t * L, L)]

    def compute(g, b):
        v_raw, v_v, w_v, dbuf = vrb[b], vb[b], wb[b], db[b]
        for t in range(C // L):
            w_v[pl.ds(t * L, L)] = jnp.exp(w_v[pl.ds(t * L, L)] - gmax)

        def edge_body(j, _):
            wvec = plsc.load_gather(w_v, [jnp.full((L,), j, jnp.int32)])
            for r in range(D // (2 * L)):
                va, vc = plsc.unpack(v_raw[j, pl.ds(r * 2 * L, 2 * L)],
                                     format=plsc.PackFormat.INTERLEAVED,
                                     preferred_element_type=jnp.float32)
                v_v[j, pl.ds(r * 2 * L, L)] = va * wvec
                v_v[j, pl.ds(r * 2 * L + L, L)] = vc * wvec
            dbuf[j, pl.ds(0, L)] = wvec
            return 0

        lax.fori_loop(0, C, edge_body, 0)

    issue_idx(0, 0)
    issue_w(0, 0)
    wait_idx(0, 0)
    issue_gather(0, 0)
    issue_idx(1, 1)
    issue_w(1, 1)

    wait_gather(0, 0)
    copy_dsc(0)
    wait_idx(1, 1)
    issue_idx(2, 0)
    issue_gather(1, 1)
    wait_w(0, 0)
    compute(0, 0)
    issue_w(2, 0)
    issue_scatter(0, 0)

    def sub_body(g, b, bo):
        wait_gather(g, b)
        copy_dsc(b)
        wait_scatter(g - 1, bo)

        @pl.when(g + 1 < NCHUNK)
        def _():
            wait_idx(g + 1, bo)
            issue_gather(g + 1, bo)

        @pl.when(g + 2 < NCHUNK)
        def _():
            issue_idx(g + 2, b)

        wait_w(g, b)
        compute(g, b)

        @pl.when(g + 2 < NCHUNK)
        def _():
            issue_w(g + 2, b)

        issue_scatter(g, b)

    def dbl_body(i, _):
        sub_body(2 * i + 1, 1, 0)
        sub_body(2 * i + 2, 0, 1)
        return 0

    lax.fori_loop(0, (NCHUNK - 1) // 2, dbl_body, 0)
    wait_scatter(NCHUNK - 1, 0)
    plsc.subcore_barrier()

    pltpu.sync_copy(msg_sh.at[pl.ds(rbase, RPT)],
                    msg_out.at[c].at[pl.ds(rbase, RPT)])
    pltpu.sync_copy(den_sh.at[pl.ds(rbase, RPT)],
                    den_out.at[c].at[pl.ds(rbase, RPT)])


def _finish_body(msg_ref, den_ref, skip_ref, o_ref):
    m = msg_ref[0] + msg_ref[1]
    dn = den_ref[0, :, 0:1] + den_ref[1, :, 0:1]
    r = m / (dn + 1e-16) + skip_ref[...]
    o_ref[...] = jnp.where(r > 0, r, jnp.exp(jnp.minimum(r, 0.0)) - 1.0)


def _finish(msgp, denp, skip):
    return pl.pallas_call(
        _finish_body,
        grid=(N // _BR,),
        in_specs=[
            pl.BlockSpec((NC, _BR, D), lambda i: (0, i, 0)),
            pl.BlockSpec((NC, _BR, L), lambda i: (0, i, 0)),
            pl.BlockSpec((_BR, D), lambda i: (i, 0)),
        ],
        out_specs=pl.BlockSpec((_BR, D), lambda i: (i, 0)),
        out_shape=jax.ShapeDtypeStruct((N, D), jnp.float32),
    )(msgp, denp, skip)


def kernel(x, edge_index, Wq, bq, Wk, bk, Wv, bv, Ws, bs):
    ei = edge_index.astype(jnp.int32)
    src = ei[0]
    dst = ei[1]
    Wv_p = Wv.reshape(D, D // 32, 2, L).transpose(0, 1, 3, 2).reshape(D, D)
    bv_p = bv.reshape(D // 32, 2, L).transpose(0, 2, 1).reshape(D)
    q, k, v, skip = _proj(x, Wq, bq, Wk, bk, Wv_p, bv_p, Ws, bs)
    alpha, gmax = _edge_logits(q, k, src.reshape(NW, NCHUNK, C),
                               dst.reshape(NW, NCHUNK, C))
    zmsg = jnp.zeros((N, D), jnp.float32)
    zden = jnp.zeros((N, L), jnp.float32)
    msgp, denp = _edge_scatter(v, src, dst, alpha, gmax, zmsg, zden)
    return _finish(msgp, denp, skip)


# speedup vs baseline: 1.3220x; 1.3220x over previous
import functools

import jax
import jax.numpy as jnp
import numpy as np
from jax import lax
from jax.experimental import pallas as pl
from jax.experimental.pallas import tpu as pltpu
from jax.experimental.pallas import tpu_sc as plsc

N = 10000
E = 320000
D = 128
NC = 2
NS = 16
NW = NC * NS
EPW = E // NW
C = 80
NCHUNK = EPW // C
L = 16
RPT = N // NS
DW = D + L

_mesh = plsc.VectorSubcoreMesh(core_axis_name="c", subcore_axis_name="s")

_GDN = lax.GatherDimensionNumbers(
    offset_dims=(), collapsed_slice_dims=(0,), start_index_map=(0,))


def _perms():
    lane = lax.iota(jnp.int32, L)
    return [jnp.reshape((lane + sh) % L, (L, 1)) for sh in (1, 2, 4, 8)]


def _shuf(x, p):
    return lax.gather(x, p, _GDN, (1,),
                      mode=lax.GatherScatterMode.PROMISE_IN_BOUNDS)


def _lane_sum(x, perms):
    for p in perms:
        x = x + _shuf(x, p)
    return x


def _lane_max(x, perms):
    for p in perms:
        x = jnp.maximum(x, _shuf(x, p))
    return x


def _proj_body(x_ref, wq, bq, wk, bk, wv, bv, ws, bs, q_ref, k_ref, v_ref, s_ref):
    xb = x_ref[...]
    scale = 1.0 / jnp.sqrt(jnp.float32(D))
    q_ref[...] = ((jnp.dot(xb, wq[...], preferred_element_type=jnp.float32)
                   + bq[...]) * scale).astype(jnp.bfloat16)
    k_ref[...] = (jnp.dot(xb, wk[...], preferred_element_type=jnp.float32)
                  + bk[...]).astype(jnp.bfloat16)
    v_ref[...] = jnp.dot(xb, wv[...], preferred_element_type=jnp.float32) + bv[...]
    s_ref[...] = jnp.dot(xb, ws[...], preferred_element_type=jnp.float32) + bs[...]


_BR = 1000


def _proj(x, Wq, bq, Wk, bk, Wv, bv, Ws, bs):
    wspec = pl.BlockSpec((D, D), lambda i: (0, 0))
    bspec = pl.BlockSpec((1, D), lambda i: (0, 0))
    rspec = pl.BlockSpec((_BR, D), lambda i: (i, 0))
    return pl.pallas_call(
        _proj_body,
        grid=(N // _BR,),
        in_specs=[rspec, wspec, bspec, wspec, bspec, wspec, bspec, wspec, bspec],
        out_specs=[rspec, rspec, rspec, rspec],
        out_shape=[jax.ShapeDtypeStruct((N, D), jnp.bfloat16),
                   jax.ShapeDtypeStruct((N, D), jnp.bfloat16),
                   jax.ShapeDtypeStruct((N, D), jnp.float32),
                   jax.ShapeDtypeStruct((N, D), jnp.float32)],
    )(x, Wq, bq.reshape(1, D), Wk, bk.reshape(1, D),
      Wv, bv.reshape(1, D), Ws, bs.reshape(1, D))


@functools.partial(
    pl.kernel,
    out_type=[jax.ShapeDtypeStruct((NW, EPW), jnp.float32),
              jax.ShapeDtypeStruct((NW, L), jnp.float32)],
    mesh=_mesh,
    scratch_types=[
        pltpu.VMEM((NCHUNK, C), jnp.int32),
        pltpu.VMEM((NCHUNK, C), jnp.int32),
        pltpu.VMEM((EPW,), jnp.float32),
        pltpu.VMEM((C, D), jnp.bfloat16),
        pltpu.VMEM((C, D), jnp.bfloat16),
        pltpu.VMEM((C, D), jnp.bfloat16),
        pltpu.VMEM((C, D), jnp.bfloat16),
        pltpu.VMEM((L,), jnp.float32),
        pltpu.SemaphoreType.DMA,
        pltpu.SemaphoreType.DMA,
        pltpu.SemaphoreType.DMA,
        pltpu.SemaphoreType.DMA,
    ],
    compiler_params=pltpu.CompilerParams(use_tc_tiling_on_sc=False,
                                         needs_layout_passes=False),
)
def _edge_logits(qh, kh, src3, dst3, alpha_h, gmax_h,
                 dst2, src2, a_all, q0, k0, q1, k1, m_v,
                 sq0, sk0, sq1, sk1):
    c = lax.axis_index("c")
    s = lax.axis_index("s")
    wid = s * NC + c

    lane = lax.iota(jnp.int32, L)
    perms = _perms()
    qb, kb = (q0, q1), (k0, k1)
    sq, sk = (sq0, sq1), (sk0, sk1)

    pltpu.sync_copy(dst3.at[wid], dst2)
    pltpu.sync_copy(src3.at[wid], src2)

    def issue(g, b):
        pltpu.async_copy(qh.at[dst2.at[g]], qb[b], sq[b])
        pltpu.async_copy(kh.at[src2.at[g]], kb[b], sk[b])

    def wait(g, b):
        pltpu.make_async_copy(qh.at[dst2.at[g]], qb[b], sq[b]).wait()
        pltpu.make_async_copy(kh.at[src2.at[g]], kb[b], sk[b]).wait()

    def compute(g, b, m16):
        q_v, k_v = qb[b], kb[b]

        def group_body(t, m16):
            av = jnp.full((L,), -3e38, jnp.float32)
            for jj in range(L):
                j = t * L + jj
                acc = None
                for r in range(D // (2 * L)):
                    qv = q_v[j, pl.ds(r * 2 * L, 2 * L)]
                    kv = k_v[j, pl.ds(r * 2 * L, 2 * L)]
                    qa, qc = plsc.unpack(qv, format=plsc.PackFormat.INTERLEAVED,
                                         preferred_element_type=jnp.float32)
                    ka, kc = plsc.unpack(kv, format=plsc.PackFormat.INTERLEAVED,
                                         preferred_element_type=jnp.float32)
                    term = qa * ka + qc * kc
                    acc = term if acc is None else acc + term
                av = jnp.where(lane == jj, _lane_sum(acc, perms), av)
            a_all[pl.ds(g * C + t * L, L)] = av
            return jnp.maximum(m16, av)

        return lax.fori_loop(0, C // L, group_body, m16)

    issue(0, 0)

    def dbl_body(i, m16):
        g = 2 * i
        wait(g, 0)
        issue(g + 1, 1)
        m16 = compute(g, 0, m16)
        wait(g + 1, 1)
        issue(g + 2, 0)
        m16 = compute(g + 1, 1, m16)
        return m16

    m16 = lax.fori_loop(0, (NCHUNK - 1) // 2, dbl_body,
                        jnp.full((L,), -3e38, jnp.float32))
    wait(NCHUNK - 1, 0)
    m16 = compute(NCHUNK - 1, 0, m16)

    pltpu.sync_copy(a_all, alpha_h.at[wid])
    m_v[...] = _lane_max(m16, perms)
    pltpu.sync_copy(m_v, gmax_h.at[wid])


@functools.partial(
    pl.kernel,
    out_type=[jax.ShapeDtypeStruct((NC, N, D), jnp.float32),
              jax.ShapeDtypeStruct((NC, N, L), jnp.float32)],
    mesh=_mesh,
    scratch_types=[
        pltpu.VMEM((C,), jnp.int32),
        pltpu.VMEM((C,), jnp.int32),
        pltpu.VMEM((C,), jnp.int32),
        pltpu.VMEM((C,), jnp.int32),
        pltpu.VMEM((C,), jnp.int32),
        pltpu.VMEM((C,), jnp.int32),
        pltpu.VMEM((C,), jnp.float32),
        pltpu.VMEM((C,), jnp.float32),
        pltpu.VMEM((C, D), jnp.float32),
        pltpu.VMEM((C, D), jnp.float32),
        pltpu.VMEM((C, L), jnp.float32),
        pltpu.VMEM((C, L), jnp.float32),
        pltpu.VMEM((NW, L), jnp.float32),
        pltpu.VMEM_SHARED((N, D), jnp.float32),
        pltpu.VMEM_SHARED((N, L), jnp.float32),
        pltpu.SemaphoreType.DMA,
        pltpu.SemaphoreType.DMA,
        pltpu.SemaphoreType.DMA,
        pltpu.SemaphoreType.DMA,
        pltpu.SemaphoreType.DMA,
        pltpu.SemaphoreType.DMA,
        pltpu.SemaphoreType.DMA,
        pltpu.SemaphoreType.DMA,
        pltpu.SemaphoreType.DMA,
        pltpu.SemaphoreType.DMA,
    ],
    compiler_params=pltpu.CompilerParams(use_tc_tiling_on_sc=False,
                                         needs_layout_passes=False),
)
def _edge_scatter(vh, srch, dsth, alphah, gmaxh, zmsgh, zdenh,
                  msg_out, den_out,
                  dst0, dst1, srcb0, srcb1, dsc0, dsc1, w0, w1,
                  v0, v1, d0, d1, gbuf, msg_sh, den_sh,
                  si0, si1, sw0, sw1, gv0, gv1, sm0, sm1, sd0, sd1):
    c = lax.axis_index("c")
    s = lax.axis_index("s")
    wid = s * NC + c
    ebase = wid * EPW
    rbase = s * RPT

    dstb, srcb, dscb = (dst0, dst1), (srcb0, srcb1), (dsc0, dsc1)
    wb, vb, db = (w0, w1), (v0, v1), (d0, d1)
    si, sw, gv = (si0, si1), (sw0, sw1), (gv0, gv1)
    sm, sd = (sm0, sm1), (sd0, sd1)

    pltpu.sync_copy(zmsgh.at[pl.ds(rbase, RPT)], msg_sh.at[pl.ds(rbase, RPT)])
    pltpu.sync_copy(zdenh.at[pl.ds(rbase, RPT)], den_sh.at[pl.ds(rbase, RPT)])

    pltpu.sync_copy(gmaxh, gbuf)
    m16 = gbuf[0, pl.ds(0, L)]
    for r in range(1, NW):
        m16 = jnp.maximum(m16, gbuf[r, pl.ds(0, L)])
    gmax = _lane_max(m16, _perms())

    plsc.subcore_barrier()

    def issue_idx(g, b):
        pltpu.async_copy(dsth.at[pl.ds(ebase + g * C, C)], dstb[b], si[b])
        pltpu.async_copy(srch.at[pl.ds(ebase + g * C, C)], srcb[b], si[b])

    def wait_idx(g, b):
        pltpu.make_async_copy(dsth.at[pl.ds(ebase + g * C, C)],
                              dstb[b], si[b]).wait()
        pltpu.make_async_copy(srch.at[pl.ds(ebase + g * C, C)],
                              srcb[b], si[b]).wait()

    def issue_w(g, b):
        pltpu.async_copy(alphah.at[wid].at[pl.ds(g * C, C)], wb[b], sw[b])

    def wait_w(g, b):
        pltpu.make_async_copy(alphah.at[wid].at[pl.ds(g * C, C)],
                              wb[b], sw[b]).wait()

    def issue_gather(g, b):
        pltpu.async_copy(vh.at[srcb[b]], vb[b], gv[b])

    def wait_gather(g, b):
        pltpu.make_async_copy(vh.at[srcb[b]], vb[b], gv[b]).wait()

    def issue_scatter(g, b):
        pltpu.async_copy(vb[b], msg_sh.at[dscb[b]], sm[b], add=True)
        pltpu.async_copy(db[b], den_sh.at[dscb[b]], sd[b], add=True)

    def wait_scatter(g, b):
        pltpu.make_async_copy(vb[b], msg_sh.at[dscb[b]], sm[b]).wait()
        pltpu.make_async_copy(db[b], den_sh.at[dscb[b]], sd[b]).wait()

    def copy_dsc(b):
        for t in range(C // L):
            dscb[b][pl.ds(t * L, L)] = dstb[b][pl.ds(t * L, L)]

    def compute(g, b):
        v_v, w_v, dbuf = vb[b], wb[b], db[b]
        for t in range(C // L):
            w_v[pl.ds(t * L, L)] = jnp.exp(w_v[pl.ds(t * L, L)] - gmax)

        def edge_body(j, _):
            wvec = plsc.load_gather(w_v, [jnp.full((L,), j, jnp.int32)])
            for r in range(D // L):
                v_v[j, pl.ds(r * L, L)] = v_v[j, pl.ds(r * L, L)] * wvec
            dbuf[j, pl.ds(0, L)] = wvec
            return 0

        lax.fori_loop(0, C, edge_body, 0)

    issue_idx(0, 0)
    issue_w(0, 0)
    wait_idx(0, 0)
    issue_gather(0, 0)
    issue_idx(1, 1)
    issue_w(1, 1)

    wait_gather(0, 0)
    copy_dsc(0)
    wait_idx(1, 1)
    issue_idx(2, 0)
    issue_gather(1, 1)
    wait_w(0, 0)
    compute(0, 0)
    issue_w(2, 0)
    issue_scatter(0, 0)

    def sub_body(g, b, bo):
        wait_gather(g, b)
        copy_dsc(b)
        wait_scatter(g - 1, bo)

        @pl.when(g + 1 < NCHUNK)
        def _():
            wait_idx(g + 1, bo)
            issue_gather(g + 1, bo)

        @pl.when(g + 2 < NCHUNK)
        def _():
            issue_idx(g + 2, b)

        wait_w(g, b)
        compute(g, b)

        @pl.when(g + 2 < NCHUNK)
        def _():
            issue_w(g + 2, b)

        issue_scatter(g, b)

    def dbl_body(i, _):
        sub_body(2 * i + 1, 1, 0)
        sub_body(2 * i + 2, 0, 1)
        return 0

    lax.fori_loop(0, (NCHUNK - 1) // 2, dbl_body, 0)
    wait_scatter(NCHUNK - 1, 0)
    plsc.subcore_barrier()

    pltpu.sync_copy(msg_sh.at[pl.ds(rbase, RPT)],
                    msg_out.at[c].at[pl.ds(rbase, RPT)])
    pltpu.sync_copy(den_sh.at[pl.ds(rbase, RPT)],
                    den_out.at[c].at[pl.ds(rbase, RPT)])


def _finish_body(msg_ref, den_ref, skip_ref, o_ref):
    m = msg_ref[0] + msg_ref[1]
    dn = den_ref[0, :, 0:1] + den_ref[1, :, 0:1]
    r = m / (dn + 1e-16) + skip_ref[...]
    o_ref[...] = jnp.where(r > 0, r, jnp.exp(jnp.minimum(r, 0.0)) - 1.0)


def _finish(msgp, denp, skip):
    return pl.pallas_call(
        _finish_body,
        grid=(N // _BR,),
        in_specs=[
            pl.BlockSpec((NC, _BR, D), lambda i: (0, i, 0)),
            pl.BlockSpec((NC, _BR, L), lambda i: (0, i, 0)),
            pl.BlockSpec((_BR, D), lambda i: (i, 0)),
        ],
        out_specs=pl.BlockSpec((_BR, D), lambda i: (i, 0)),
        out_shape=jax.ShapeDtypeStruct((N, D), jnp.float32),
    )(msgp, denp, skip)


def kernel(x, edge_index, Wq, bq, Wk, bk, Wv, bv, Ws, bs):
    ei = edge_index.astype(jnp.int32)
    src = ei[0]
    dst = ei[1]
    q, k, v, skip = _proj(x, Wq, bq, Wk, bk, Wv, bv, Ws, bs)
    alpha, gmax = _edge_logits(q, k, src.reshape(NW, NCHUNK, C),
                               dst.reshape(NW, NCHUNK, C))
    zmsg = jnp.zeros((N, D), jnp.float32)
    zden = jnp.zeros((N, L), jnp.float32)
    msgp, denp = _edge_scatter(v, src, dst, alpha, gmax, zmsg, zden)
    return _finish(msgp, denp, skip)


# speedup vs baseline: 1.4062x; 1.0637x over previous
import functools

import jax
import jax.numpy as jnp
import numpy as np
from jax import lax
from jax.experimental import pallas as pl
from jax.experimental.pallas import tpu as pltpu
from jax.experimental.pallas import tpu_sc as plsc

N = 10000
E = 320000
D = 128
NC = 2
NS = 16
NW = NC * NS
EPW = E // NW
C = 80
NCHUNK = EPW // C
L = 16
RPT = N // NS
DW = D + L

_mesh = plsc.VectorSubcoreMesh(core_axis_name="c", subcore_axis_name="s")

_GDN = lax.GatherDimensionNumbers(
    offset_dims=(), collapsed_slice_dims=(0,), start_index_map=(0,))


def _perms():
    lane = lax.iota(jnp.int32, L)
    return [jnp.reshape((lane + sh) % L, (L, 1)) for sh in (1, 2, 4, 8)]


def _shuf(x, p):
    return lax.gather(x, p, _GDN, (1,),
                      mode=lax.GatherScatterMode.PROMISE_IN_BOUNDS)


def _lane_sum(x, perms):
    for p in perms:
        x = x + _shuf(x, p)
    return x


def _lane_max(x, perms):
    for p in perms:
        x = jnp.maximum(x, _shuf(x, p))
    return x


def _proj_body(x_ref, wq, bq, wk, bk, wv, bv, ws, bs, q_ref, k_ref, v_ref, s_ref):
    xb = x_ref[...]
    scale = 1.0 / jnp.sqrt(jnp.float32(D))
    q_ref[...] = ((jnp.dot(xb, wq[...], preferred_element_type=jnp.float32)
                   + bq[...]) * scale).astype(jnp.bfloat16)
    k_ref[...] = (jnp.dot(xb, wk[...], preferred_element_type=jnp.float32)
                  + bk[...]).astype(jnp.bfloat16)
    v_ref[...] = jnp.dot(xb, wv[...], preferred_element_type=jnp.float32) + bv[...]
    s_ref[...] = jnp.dot(xb, ws[...], preferred_element_type=jnp.float32) + bs[...]


_BR = 1000


def _proj(x, Wq, bq, Wk, bk, Wv, bv, Ws, bs):
    wspec = pl.BlockSpec((D, D), lambda i: (0, 0))
    bspec = pl.BlockSpec((1, D), lambda i: (0, 0))
    rspec = pl.BlockSpec((_BR, D), lambda i: (i, 0))
    return pl.pallas_call(
        _proj_body,
        grid=(N // _BR,),
        in_specs=[rspec, wspec, bspec, wspec, bspec, wspec, bspec, wspec, bspec],
        out_specs=[rspec, rspec, rspec, rspec],
        out_shape=[jax.ShapeDtypeStruct((N, D), jnp.bfloat16),
                   jax.ShapeDtypeStruct((N, D), jnp.bfloat16),
                   jax.ShapeDtypeStruct((N, D), jnp.float32),
                   jax.ShapeDtypeStruct((N, D), jnp.float32)],
    )(x, Wq, bq.reshape(1, D), Wk, bk.reshape(1, D),
      Wv, bv.reshape(1, D), Ws, bs.reshape(1, D))


@functools.partial(
    pl.kernel,
    out_type=[jax.ShapeDtypeStruct((NW, EPW), jnp.float32),
              jax.ShapeDtypeStruct((NW, L), jnp.float32)],
    mesh=_mesh,
    scratch_types=[
        pltpu.VMEM((NCHUNK, C), jnp.int32),
        pltpu.VMEM((NCHUNK, C), jnp.int32),
        pltpu.VMEM((EPW,), jnp.float32),
        pltpu.VMEM((C, D), jnp.bfloat16),
        pltpu.VMEM((C, D), jnp.bfloat16),
        pltpu.VMEM((C, D), jnp.bfloat16),
        pltpu.VMEM((C, D), jnp.bfloat16),
        pltpu.VMEM((L,), jnp.float32),
        pltpu.SemaphoreType.DMA,
        pltpu.SemaphoreType.DMA,
        pltpu.SemaphoreType.DMA,
        pltpu.SemaphoreType.DMA,
    ],
    compiler_params=pltpu.CompilerParams(use_tc_tiling_on_sc=False,
                                         needs_layout_passes=False),
)
def _edge_logits(qh, kh, src3, dst3, alpha_h, gmax_h,
                 dst2, src2, a_all, q0, k0, q1, k1, m_v,
                 sq0, sk0, sq1, sk1):
    c = lax.axis_index("c")
    s = lax.axis_index("s")
    wid = s * NC + c

    lane = lax.iota(jnp.int32, L)
    perms = _perms()
    qb, kb = (q0, q1), (k0, k1)
    sq, sk = (sq0, sq1), (sk0, sk1)

    pltpu.sync_copy(dst3.at[wid], dst2)
    pltpu.sync_copy(src3.at[wid], src2)

    def issue(g, b):
        pltpu.async_copy(qh.at[dst2.at[g]], qb[b], sq[b])
        pltpu.async_copy(kh.at[src2.at[g]], kb[b], sk[b])

    def wait(g, b):
        pltpu.make_async_copy(qh.at[dst2.at[g]], qb[b], sq[b]).wait()
        pltpu.make_async_copy(kh.at[src2.at[g]], kb[b], sk[b]).wait()

    def compute(g, b, m16):
        q_v, k_v = qb[b], kb[b]

        def group_body(t, m16):
            av = jnp.full((L,), -3e38, jnp.float32)
            for jj in range(L):
                j = t * L + jj
                acc = None
                for r in range(D // (2 * L)):
                    qv = q_v[j, pl.ds(r * 2 * L, 2 * L)]
                    kv = k_v[j, pl.ds(r * 2 * L, 2 * L)]
                    pa, pc = plsc.unpack(qv * kv,
                                         format=plsc.PackFormat.INTERLEAVED,
                                         preferred_element_type=jnp.float32)
                    term = pa + pc
                    acc = term if acc is None else acc + term
                av = jnp.where(lane == jj, _lane_sum(acc, perms), av)
            a_all[pl.ds(g * C + t * L, L)] = av
            return jnp.maximum(m16, av)

        return lax.fori_loop(0, C // L, group_body, m16)

    issue(0, 0)

    def dbl_body(i, m16):
        g = 2 * i
        wait(g, 0)
        issue(g + 1, 1)
        m16 = compute(g, 0, m16)
        wait(g + 1, 1)
        issue(g + 2, 0)
        m16 = compute(g + 1, 1, m16)
        return m16

    m16 = lax.fori_loop(0, (NCHUNK - 1) // 2, dbl_body,
                        jnp.full((L,), -3e38, jnp.float32))
    wait(NCHUNK - 1, 0)
    m16 = compute(NCHUNK - 1, 0, m16)

    pltpu.sync_copy(a_all, alpha_h.at[wid])
    m_v[...] = _lane_max(m16, perms)
    pltpu.sync_copy(m_v, gmax_h.at[wid])


@functools.partial(
    pl.kernel,
    out_type=[jax.ShapeDtypeStruct((NC, N, D), jnp.float32),
              jax.ShapeDtypeStruct((NC, N, L), jnp.float32)],
    mesh=_mesh,
    scratch_types=[
        pltpu.VMEM((C,), jnp.int32),
        pltpu.VMEM((C,), jnp.int32),
        pltpu.VMEM((C,), jnp.int32),
        pltpu.VMEM((C,), jnp.int32),
        pltpu.VMEM((C,), jnp.int32),
        pltpu.VMEM((C,), jnp.int32),
        pltpu.VMEM((C,), jnp.float32),
        pltpu.VMEM((C,), jnp.float32),
        pltpu.VMEM((C, D), jnp.float32),
        pltpu.VMEM((C, D), jnp.float32),
        pltpu.VMEM((C, L), jnp.float32),
        pltpu.VMEM((C, L), jnp.float32),
        pltpu.VMEM((NW, L), jnp.float32),
        pltpu.VMEM_SHARED((N, D), jnp.float32),
        pltpu.VMEM_SHARED((N, L), jnp.float32),
        pltpu.SemaphoreType.DMA,
        pltpu.SemaphoreType.DMA,
        pltpu.SemaphoreType.DMA,
        pltpu.SemaphoreType.DMA,
        pltpu.SemaphoreType.DMA,
        pltpu.SemaphoreType.DMA,
        pltpu.SemaphoreType.DMA,
        pltpu.SemaphoreType.DMA,
        pltpu.SemaphoreType.DMA,
        pltpu.SemaphoreType.DMA,
    ],
    compiler_params=pltpu.CompilerParams(use_tc_tiling_on_sc=False,
                                         needs_layout_passes=False),
)
def _edge_scatter(vh, srch, dsth, alphah, gmaxh, zmsgh, zdenh,
                  msg_out, den_out,
                  dst0, dst1, srcb0, srcb1, dsc0, dsc1, w0, w1,
                  v0, v1, d0, d1, gbuf, msg_sh, den_sh,
                  si0, si1, sw0, sw1, gv0, gv1, sm0, sm1, sd0, sd1):
    c = lax.axis_index("c")
    s = lax.axis_index("s")
    wid = s * NC + c
    ebase = wid * EPW
    rbase = s * RPT

    dstb, srcb, dscb = (dst0, dst1), (srcb0, srcb1), (dsc0, dsc1)
    wb, vb, db = (w0, w1), (v0, v1), (d0, d1)
    si, sw, gv = (si0, si1), (sw0, sw1), (gv0, gv1)
    sm, sd = (sm0, sm1), (sd0, sd1)

    pltpu.sync_copy(zmsgh.at[pl.ds(rbase, RPT)], msg_sh.at[pl.ds(rbase, RPT)])
    pltpu.sync_copy(zdenh.at[pl.ds(rbase, RPT)], den_sh.at[pl.ds(rbase, RPT)])

    pltpu.sync_copy(gmaxh, gbuf)
    m16 = gbuf[0, pl.ds(0, L)]
    for r in range(1, NW):
        m16 = jnp.maximum(m16, gbuf[r, pl.ds(0, L)])
    gmax = _lane_max(m16, _perms())

    plsc.subcore_barrier()

    def issue_idx(g, b):
        pltpu.async_copy(dsth.at[pl.ds(ebase + g * C, C)], dstb[b], si[b])
        pltpu.async_copy(srch.at[pl.ds(ebase + g * C, C)], srcb[b], si[b])

    def wait_idx(g, b):
        pltpu.make_async_copy(dsth.at[pl.ds(ebase + g * C, C)],
                              dstb[b], si[b]).wait()
        pltpu.make_async_copy(srch.at[pl.ds(ebase + g * C, C)],
                              srcb[b], si[b]).wait()

    def issue_w(g, b):
        pltpu.async_copy(alphah.at[wid].at[pl.ds(g * C, C)], wb[b], sw[b])

    def wait_w(g, b):
        pltpu.make_async_copy(alphah.at[wid].at[pl.ds(g * C, C)],
                              wb[b], sw[b]).wait()

    def issue_gather(g, b):
        pltpu.async_copy(vh.at[srcb[b]], vb[b], gv[b])

    def wait_gather(g, b):
        pltpu.make_async_copy(vh.at[srcb[b]], vb[b], gv[b]).wait()

    def issue_scatter(g, b):
        pltpu.async_copy(vb[b], msg_sh.at[dscb[b]], sm[b], add=True)
        pltpu.async_copy(db[b], den_sh.at[dscb[b]], sd[b], add=True)

    def wait_scatter(g, b):
        pltpu.make_async_copy(vb[b], msg_sh.at[dscb[b]], sm[b]).wait()
        pltpu.make_async_copy(db[b], den_sh.at[dscb[b]], sd[b]).wait()

    def copy_dsc(b):
        for t in range(C // L):
            dscb[b][pl.ds(t * L, L)] = dstb[b][pl.ds(t * L, L)]

    lane = lax.iota(jnp.int32, L)
    bcast = [jnp.reshape(lane * 0 + jj, (L, 1)) for jj in range(L)]

    def compute(g, b):
        v_v, w_v, dbuf = vb[b], wb[b], db[b]
        for t in range(C // L):
            w_v[pl.ds(t * L, L)] = jnp.exp(w_v[pl.ds(t * L, L)] - gmax)

        def group_body(t, _):
            w16 = w_v[pl.ds(t * L, L)]
            for jj in range(L):
                j = t * L + jj
                wvec = _shuf(w16, bcast[jj])
                for r in range(D // L):
                    v_v[j, pl.ds(r * L, L)] = v_v[j, pl.ds(r * L, L)] * wvec
                dbuf[j, pl.ds(0, L)] = wvec
            return 0

        lax.fori_loop(0, C // L, group_body, 0)

    issue_idx(0, 0)
    issue_w(0, 0)
    wait_idx(0, 0)
    issue_gather(0, 0)
    issue_idx(1, 1)
    issue_w(1, 1)

    wait_gather(0, 0)
    copy_dsc(0)
    wait_idx(1, 1)
    issue_idx(2, 0)
    issue_gather(1, 1)
    wait_w(0, 0)
    compute(0, 0)
    issue_w(2, 0)
    issue_scatter(0, 0)

    def sub_body(g, b, bo):
        wait_gather(g, b)
        copy_dsc(b)
        wait_scatter(g - 1, bo)

        @pl.when(g + 1 < NCHUNK)
        def _():
            wait_idx(g + 1, bo)
            issue_gather(g + 1, bo)

        @pl.when(g + 2 < NCHUNK)
        def _():
            issue_idx(g + 2, b)

        wait_w(g, b)
        compute(g, b)

        @pl.when(g + 2 < NCHUNK)
        def _():
            issue_w(g + 2, b)

        issue_scatter(g, b)

    def dbl_body(i, _):
        sub_body(2 * i + 1, 1, 0)
        sub_body(2 * i + 2, 0, 1)
        return 0

    lax.fori_loop(0, (NCHUNK - 1) // 2, dbl_body, 0)
    wait_scatter(NCHUNK - 1, 0)
    plsc.subcore_barrier()

    pltpu.sync_copy(msg_sh.at[pl.ds(rbase, RPT)],
                    msg_out.at[c].at[pl.ds(rbase, RPT)])
    pltpu.sync_copy(den_sh.at[pl.ds(rbase, RPT)],
                    den_out.at[c].at[pl.ds(rbase, RPT)])


def _finish_body(msg_ref, den_ref, skip_ref, o_ref):
    m = msg_ref[0] + msg_ref[1]
    dn = den_ref[0, :, 0:1] + den_ref[1, :, 0:1]
    r = m / (dn + 1e-16) + skip_ref[...]
    o_ref[...] = jnp.where(r > 0, r, jnp.exp(jnp.minimum(r, 0.0)) - 1.0)


def _finish(msgp, denp, skip):
    return pl.pallas_call(
        _finish_body,
        grid=(N // _BR,),
        in_specs=[
            pl.BlockSpec((NC, _BR, D), lambda i: (0, i, 0)),
            pl.BlockSpec((NC, _BR, L), lambda i: (0, i, 0)),
            pl.BlockSpec((_BR, D), lambda i: (i, 0)),
        ],
        out_specs=pl.BlockSpec((_BR, D), lambda i: (i, 0)),
        out_shape=jax.ShapeDtypeStruct((N, D), jnp.float32),
    )(msgp, denp, skip)


def kernel(x, edge_index, Wq, bq, Wk, bk, Wv, bv, Ws, bs):
    ei = edge_index.astype(jnp.int32)
    src = ei[0]
    dst = ei[1]
    q, k, v, skip = _proj(x, Wq, bq, Wk, bk, Wv, bv, Ws, bs)
    alpha, gmax = _edge_logits(q, k, src.reshape(NW, NCHUNK, C),
                               dst.reshape(NW, NCHUNK, C))
    zmsg = jnp.zeros((N, D), jnp.float32)
    zden = jnp.zeros((N, L), jnp.float32)
    msgp, denp = _edge_scatter(v, src, dst, alpha, gmax, zmsg, zden)
    return _finish(msgp, denp, skip)
